# Initial kernel scaffold; baseline (speedup 1.0000x reference)
import jax
import jax.numpy as jnp
from jax.experimental import pallas as pl


def kernel(enc_r1, enc_w1, enc_b1, enc_r2, enc_w2, enc_b2, pool_re, pool_ro, pool_ce, pool_co, fc_we, fc_be, fc_wml, fc_bml, fc_wst1, fc_bst1, fc_wst2, fc_bst2, fc_wst3, fc_bst3, fc_wp, fc_bp, fc_wd1, fc_bd1, fc_wd2, fc_bd2, dec_r1, dec_w1, dec_b1, dec_r2, dec_w2, dec_b2, x_nchw):
    raise NotImplementedError("write your pallas kernel here")


# speedup vs baseline: 2.0140x; 2.0140x over previous
import math

import jax
import jax.numpy as jnp
from jax.experimental import pallas as pl
from jax.experimental.pallas import tpu as pltpu

_VMEM_LIMIT = 100 * 1024 * 1024
_G = 2


def _dot(a, b):
    return jnp.dot(a, b, preferred_element_type=jnp.float32)


def _relu(v):
    return jnp.maximum(v, 0.0)


def _enc_kernel(x_ref, w1_ref, b1_ref, w2_ref, b2_ref, ce_ref, co_ref,
                out_ref, h1_scr, h2_scr):
    K = w1_ref.shape[0]
    Ho1 = h1_scr.shape[0]
    Ho2 = h2_scr.shape[0]

    for io in range(Ho1):
        acc = _dot(x_ref[io], w1_ref[0])
        for di in range(1, K):
            acc = acc + _dot(x_ref[io + di], w1_ref[di])
        h1_scr[io] = _relu(acc + b1_ref[...])

    for io in range(Ho2):
        acc = _dot(h1_scr[io], w2_ref[0])
        for di in range(1, K):
            acc = acc + _dot(h1_scr[io + di], w2_ref[di])
        h2_scr[io] = _relu(acc + b2_ref[...])

    Wp = ce_ref.shape[1]
    for i in range(Ho2 // 2):
        ph = jnp.maximum(h2_scr[2 * i], h2_scr[2 * i + 1])
        pooled = jnp.maximum(_dot(ph, ce_ref[...]), _dot(ph, co_ref[...]))
        out_ref[:, i * Wp:(i + 1) * Wp] = pooled


def _enc_call(xT, w1, b1, w2, b2, ce, co):
    H, B, WC = xT.shape
    Bc = B // _G
    Ho1 = H - w1.shape[0] + 1
    Ho2 = Ho1 - w2.shape[0] + 1
    Wp = ce.shape[1]
    n1 = w1.shape[1]
    del n1
    return pl.pallas_call(
        _enc_kernel,
        grid=(_G,),
        out_shape=jax.ShapeDtypeStruct((B, (Ho2 // 2) * Wp), jnp.float32),
        in_specs=[
            pl.BlockSpec((H, Bc, WC), lambda g: (0, g, 0)),
            pl.BlockSpec(w1.shape, lambda g: (0, 0, 0)),
            pl.BlockSpec(b1.shape, lambda g: (0, 0)),
            pl.BlockSpec(w2.shape, lambda g: (0, 0, 0)),
            pl.BlockSpec(b2.shape, lambda g: (0, 0)),
            pl.BlockSpec(ce.shape, lambda g: (0, 0)),
            pl.BlockSpec(co.shape, lambda g: (0, 0)),
        ],
        out_specs=pl.BlockSpec((Bc, (Ho2 // 2) * Wp), lambda g: (g, 0)),
        scratch_shapes=[
            pltpu.VMEM((Ho1, Bc, w1.shape[2]), jnp.float32),
            pltpu.VMEM((Ho2, Bc, w2.shape[2]), jnp.float32),
        ],
        compiler_params=pltpu.CompilerParams(
            dimension_semantics=("parallel",), vmem_limit_bytes=_VMEM_LIMIT),
    )(xT, w1, b1, w2, b2, ce, co)


def _dense_kernel(hf_ref, we_ref, be_ref, wml_ref, bml_ref,
                  w1_ref, b1_ref, w2_ref, b2_ref, w3_ref, b3_ref,
                  wp_ref, bp_ref, wd1_ref, bd1_ref, wd2_ref, bd2_ref,
                  d2_ref, ldj_ref, lpz_ref):
    L = wml_ref.shape[1] // 2

    h = _relu(_dot(hf_ref[...], we_ref[...]) + be_ref[...])
    ml = _dot(h, wml_ref[...]) + bml_ref[...]
    xa, xb = ml[:, :L], ml[:, L:]

    st = _relu(_dot(xa, w1_ref[...]) + b1_ref[...])
    st = _relu(_dot(st, w2_ref[...]) + b2_ref[...])
    st = _dot(st, w3_ref[...]) + b3_ref[...]
    s = _relu(st[:, :L])
    t = st[:, L:]

    yb = (xb - t) * jnp.exp(-s)
    ldj_ref[...] = -jnp.sum(s, axis=1, keepdims=True)

    z = _relu(_dot(xa, wp_ref[:L, :]) + _dot(yb, wp_ref[L:, :]) + bp_ref[...])
    lpz_ref[...] = (-0.5 * jnp.sum(z * z, axis=1, keepdims=True)
                    - 0.5 * z.shape[1] * math.log(2.0 * math.pi))

    d = _relu(_dot(z, wd1_ref[...]) + bd1_ref[...])
    d2_ref[...] = _relu(_dot(d, wd2_ref[...]) + bd2_ref[...])


def _dense_call(hf, we, be, wml, bml, w1, b1, w2, b2, w3, b3,
                wp, bp, wd1, bd1, wd2, bd2):
    B, F = hf.shape
    Bc = B // _G
    D_out = wd2.shape[1]
    ws = (we, be, wml, bml, w1, b1, w2, b2, w3, b3, wp, bp, wd1, bd1, wd2, bd2)
    in_specs = [pl.BlockSpec((Bc, F), lambda g: (g, 0))] + \
               [pl.BlockSpec(w.shape, lambda g: (0, 0)) for w in ws]
    return pl.pallas_call(
        _dense_kernel,
        grid=(_G,),
        out_shape=(jax.ShapeDtypeStruct((B, D_out), jnp.float32),
                   jax.ShapeDtypeStruct((B, 1), jnp.float32),
                   jax.ShapeDtypeStruct((B, 1), jnp.float32)),
        in_specs=in_specs,
        out_specs=(pl.BlockSpec((Bc, D_out), lambda g: (g, 0)),
                   pl.BlockSpec((Bc, 1), lambda g: (g, 0)),
                   pl.BlockSpec((Bc, 1), lambda g: (g, 0))),
        compiler_params=pltpu.CompilerParams(
            dimension_semantics=("parallel",), vmem_limit_bytes=_VMEM_LIMIT),
    )(hf, *ws)


def _dec_kernel(d2_ref, w1_ref, b1_ref, w2_ref, b2_ref, out_ref, h_scr):
    K = w1_ref.shape[0]
    P = K - 1
    Hin = d2_ref.shape[1] // w1_ref.shape[1]
    Ho1 = h_scr.shape[0]
    Ho2 = out_ref.shape[0]
    WCi = w1_ref.shape[1]

    for io in range(Ho1):
        acc = None
        for di in range(K):
            h = io + di - P
            if 0 <= h < Hin:
                term = _dot(d2_ref[:, h * WCi:(h + 1) * WCi], w1_ref[di])
                acc = term if acc is None else acc + term
        h_scr[io] = _relu(acc + b1_ref[...])

    for io in range(Ho2):
        acc = None
        for di in range(K):
            h = io + di - P
            if 0 <= h < Ho1:
                term = _dot(h_scr[h], w2_ref[di])
                acc = term if acc is None else acc + term
        out_ref[io] = acc + b2_ref[...]


def _dec_call(d2, w1, b1, w2, b2):
    B, D = d2.shape
    Bc = B // _G
    K = w1.shape[0]
    P = K - 1
    Hin = D // w1.shape[1]
    Ho1 = Hin + 2 * P - K + 1
    Ho2 = Ho1 + 2 * P - K + 1
    WCo = w2.shape[2]
    return pl.pallas_call(
        _dec_kernel,
        grid=(_G,),
        out_shape=jax.ShapeDtypeStruct((Ho2, B, WCo), jnp.float32),
        in_specs=[
            pl.BlockSpec((Bc, D), lambda g: (g, 0)),
            pl.BlockSpec(w1.shape, lambda g: (0, 0, 0)),
            pl.BlockSpec(b1.shape, lambda g: (0, 0)),
            pl.BlockSpec(w2.shape, lambda g: (0, 0, 0)),
            pl.BlockSpec(b2.shape, lambda g: (0, 0)),
        ],
        out_specs=pl.BlockSpec((Ho2, Bc, WCo), lambda g: (0, g, 0)),
        scratch_shapes=[
            pltpu.VMEM((Ho1, Bc, w1.shape[2]), jnp.float32),
        ],
        compiler_params=pltpu.CompilerParams(
            dimension_semantics=("parallel",), vmem_limit_bytes=_VMEM_LIMIT),
    )(d2, w1, b1, w2, b2)


def kernel(enc_r1, enc_w1, enc_b1, enc_r2, enc_w2, enc_b2,
           pool_re, pool_ro, pool_ce, pool_co,
           fc_we, fc_be, fc_wml, fc_bml, fc_wst1, fc_bst1,
           fc_wst2, fc_bst2, fc_wst3, fc_bst3, fc_wp, fc_bp,
           fc_wd1, fc_bd1, fc_wd2, fc_bd2,
           dec_r1, dec_w1, dec_b1, dec_r2, dec_w2, dec_b2,
           x_nchw):
    B, C, H, W = x_nchw.shape

    xT = jnp.transpose(x_nchw.astype(jnp.float32), (2, 0, 3, 1)).reshape(H, B, W * C)

    hf = _enc_call(xT, enc_w1, enc_b1, enc_w2, enc_b2, pool_ce, pool_co)

    d2, ldj, lpz = _dense_call(hf, fc_we, fc_be, fc_wml, fc_bml,
                               fc_wst1, fc_bst1, fc_wst2, fc_bst2,
                               fc_wst3, fc_bst3, fc_wp, fc_bp,
                               fc_wd1, fc_bd1, fc_wd2, fc_bd2)

    dec = _dec_call(d2, dec_w1, dec_b1, dec_w2, dec_b2)

    x_hat = jnp.transpose(dec.reshape(H, B, W, C), (1, 3, 0, 2))
    return x_hat, ldj[:, 0], lpz[:, 0]


# speedup vs baseline: 2.1390x; 1.0621x over previous
import math

import jax
import jax.numpy as jnp
from jax.experimental import pallas as pl
from jax.experimental.pallas import tpu as pltpu

_VMEM_LIMIT = 100 * 1024 * 1024
_G = 2


def _dot(a, b):
    return jnp.dot(a, b, preferred_element_type=jnp.float32)


def _relu(v):
    return jnp.maximum(v, 0.0)


def _fused_kernel(x_ref, w1_ref, b1_ref, w2_ref, b2_ref,
                  we_ref, be_ref, wml_ref, bml_ref,
                  ws1_ref, bs1_ref, ws2_ref, bs2_ref, ws3_ref, bs3_ref,
                  wp_ref, bp_ref, wd1_ref, bd1_ref, wd2_ref, bd2_ref,
                  dw1_ref, db1_ref, dw2_ref, db2_ref,
                  dec_ref, ldj_ref, lpz_ref,
                  h1_scr, h2_scr, d2_scr, hd_scr):
    K = w1_ref.shape[0]
    Ho1 = h1_scr.shape[0]
    Ho2 = h2_scr.shape[0]
    NF2 = 64

    for io in range(Ho1):
        acc = _dot(x_ref[io], w1_ref[0])
        for di in range(1, K):
            acc = acc + _dot(x_ref[io + di], w1_ref[di])
        h1_scr[io] = _relu(acc + b1_ref[...])

    for io in range(Ho2):
        acc = _dot(h1_scr[io], w2_ref[0])
        for di in range(1, K):
            acc = acc + _dot(h1_scr[io + di], w2_ref[di])
        h2_scr[io] = _relu(acc + b2_ref[...])

    Wp = Ho2 // 2
    pieces = []
    for i in range(Wp):
        ph = jnp.maximum(h2_scr[2 * i], h2_scr[2 * i + 1])
        for j in range(Wp):
            pieces.append(jnp.maximum(ph[:, j * 2 * NF2:(j * 2 + 1) * NF2],
                                      ph[:, (j * 2 + 1) * NF2:(j + 1) * 2 * NF2]))
    hf = jnp.concatenate(pieces, axis=1)

    L = wml_ref.shape[1] // 2

    h = _relu(_dot(hf, we_ref[...]) + be_ref[...])
    ml = _dot(h, wml_ref[...]) + bml_ref[...]
    xa, xb = ml[:, :L], ml[:, L:]

    st = _relu(_dot(xa, ws1_ref[...]) + bs1_ref[...])
    st = _relu(_dot(st, ws2_ref[...]) + bs2_ref[...])
    st = _dot(st, ws3_ref[...]) + bs3_ref[...]
    s = _relu(st[:, :L])
    t = st[:, L:]

    yb = (xb - t) * jnp.exp(-s)
    ldj_ref[...] = -jnp.sum(s, axis=1, keepdims=True)

    z = _relu(_dot(xa, wp_ref[:L, :]) + _dot(yb, wp_ref[L:, :]) + bp_ref[...])
    lpz_ref[...] = (-0.5 * jnp.sum(z * z, axis=1, keepdims=True)
                    - 0.5 * z.shape[1] * math.log(2.0 * math.pi))

    d = _relu(_dot(z, wd1_ref[...]) + bd1_ref[...])
    d2_scr[...] = _relu(_dot(d, wd2_ref[...]) + bd2_ref[...])

    P = K - 1
    WCi = dw1_ref.shape[1]
    Hd1 = hd_scr.shape[0]
    Hd2 = dec_ref.shape[0]

    for io in range(Hd1):
        acc = None
        for di in range(K):
            hrow = io + di - P
            if 0 <= hrow < Ho2:
                term = _dot(d2_scr[:, hrow * WCi:(hrow + 1) * WCi], dw1_ref[di])
                acc = term if acc is None else acc + term
        hd_scr[io] = _relu(acc + db1_ref[...])

    for io in range(Hd2):
        acc = None
        for di in range(K):
            hrow = io + di - P
            if 0 <= hrow < Hd1:
                term = _dot(hd_scr[hrow], dw2_ref[di])
                acc = term if acc is None else acc + term
        dec_ref[io] = acc + db2_ref[...]


def kernel(enc_r1, enc_w1, enc_b1, enc_r2, enc_w2, enc_b2,
           pool_re, pool_ro, pool_ce, pool_co,
           fc_we, fc_be, fc_wml, fc_bml, fc_wst1, fc_bst1,
           fc_wst2, fc_bst2, fc_wst3, fc_bst3, fc_wp, fc_bp,
           fc_wd1, fc_bd1, fc_wd2, fc_bd2,
           dec_r1, dec_w1, dec_b1, dec_r2, dec_w2, dec_b2,
           x_nchw):
    B, C, H, W = x_nchw.shape
    Bc = B // _G
    K = enc_w1.shape[0]
    Ho1 = H - K + 1
    Ho2 = Ho1 - K + 1
    D_out = fc_wd2.shape[1]

    xT = jnp.transpose(x_nchw.astype(jnp.float32), (2, 0, 3, 1)).reshape(H, B, W * C)

    ws = (enc_w1, enc_b1, enc_w2, enc_b2,
          fc_we, fc_be, fc_wml, fc_bml, fc_wst1, fc_bst1,
          fc_wst2, fc_bst2, fc_wst3, fc_bst3, fc_wp, fc_bp,
          fc_wd1, fc_bd1, fc_wd2, fc_bd2,
          dec_w1, dec_b1, dec_w2, dec_b2)

    in_specs = [pl.BlockSpec((H, Bc, W * C), lambda g: (0, g, 0))] + \
               [pl.BlockSpec(w.shape, (lambda n: (lambda g: (0,) * n))(w.ndim))
                for w in ws]

    dec, ldj, lpz = pl.pallas_call(
        _fused_kernel,
        grid=(_G,),
        out_shape=(jax.ShapeDtypeStruct((H, B, W * C), jnp.float32),
                   jax.ShapeDtypeStruct((B, 1), jnp.float32),
                   jax.ShapeDtypeStruct((B, 1), jnp.float32)),
        in_specs=in_specs,
        out_specs=(pl.BlockSpec((H, Bc, W * C), lambda g: (0, g, 0)),
                   pl.BlockSpec((Bc, 1), lambda g: (g, 0)),
                   pl.BlockSpec((Bc, 1), lambda g: (g, 0))),
        scratch_shapes=[
            pltpu.VMEM((Ho1, Bc, enc_w1.shape[2]), jnp.float32),
            pltpu.VMEM((Ho2, Bc, enc_w2.shape[2]), jnp.float32),
            pltpu.VMEM((Bc, D_out), jnp.float32),
            pltpu.VMEM((Ho1, Bc, dec_w1.shape[2]), jnp.float32),
        ],
        compiler_params=pltpu.CompilerParams(
            dimension_semantics=("parallel",), vmem_limit_bytes=_VMEM_LIMIT),
    )(xT, *ws)

    x_hat = jnp.transpose(dec.reshape(H, B, W, C), (1, 3, 0, 2))
    return x_hat, ldj[:, 0], lpz[:, 0]


# speedup vs baseline: 2.2456x; 1.0498x over previous
import math

import jax
import jax.numpy as jnp
from jax.experimental import pallas as pl
from jax.experimental.pallas import tpu as pltpu

_VMEM_LIMIT = 100 * 1024 * 1024
_G = 2


def _dot(a, b):
    return jnp.dot(a, b, preferred_element_type=jnp.float32)


def _relu(v):
    return jnp.maximum(v, 0.0)


def _fused_kernel(x_ref, w1_ref, b1_ref, c2_ref, b2_ref,
                  we_ref, be_ref, wml_ref, bml_ref,
                  ws1_ref, bs1_ref, ws2_ref, bs2_ref, ws3_ref, bs3_ref,
                  wp_ref, bp_ref, wd1_ref, bd1_ref, wd2_ref, bd2_ref,
                  c1_ref, db1_ref, dw2_ref, db2_ref,
                  dec_ref, ldj_ref, lpz_ref,
                  h1_scr, h2_scr, d2_scr, hd_scr, w2b_scr, w1b_scr):
    K = w1_ref.shape[0]
    Ho1 = h1_scr.shape[0]
    Ho2 = h2_scr.shape[0]
    NF2 = 64

    NF1 = 32
    w2b_scr[...] = jnp.zeros_like(w2b_scr)
    for w in range(Ho2):
        w2b_scr[:, w * NF1:(w + K) * NF1, w * NF2:(w + 1) * NF2] = \
            c2_ref[:, :, w * NF2:(w + 1) * NF2]
    Hd2 = dec_ref.shape[0]
    w1b_scr[...] = jnp.zeros_like(w1b_scr)
    for o in range(Ho1):
        s = min(max(o - 2, 0), Ho2 - K) * NF2
        w1b_scr[:, s:s + K * NF2, o * NF1:(o + 1) * NF1] = \
            c1_ref[:, :, o * NF1:(o + 1) * NF1]

    for io in range(Ho1):
        acc = _dot(x_ref[io], w1_ref[0])
        for di in range(1, K):
            acc = acc + _dot(x_ref[io + di], w1_ref[di])
        h1_scr[io] = _relu(acc + b1_ref[...])

    for io in range(Ho2):
        acc = _dot(h1_scr[io], w2b_scr[0])
        for di in range(1, K):
            acc = acc + _dot(h1_scr[io + di], w2b_scr[di])
        h2_scr[io] = _relu(acc + b2_ref[...])

    Wp = Ho2 // 2
    pieces = []
    for i in range(Wp):
        ph = jnp.maximum(h2_scr[2 * i], h2_scr[2 * i + 1])
        for j in range(Wp):
            pieces.append(jnp.maximum(ph[:, j * 2 * NF2:(j * 2 + 1) * NF2],
                                      ph[:, (j * 2 + 1) * NF2:(j + 1) * 2 * NF2]))
    hf = jnp.concatenate(pieces, axis=1)

    L = wml_ref.shape[1] // 2

    h = _relu(_dot(hf, we_ref[...]) + be_ref[...])
    ml = _dot(h, wml_ref[...]) + bml_ref[...]
    xa, xb = ml[:, :L], ml[:, L:]

    st = _relu(_dot(xa, ws1_ref[...]) + bs1_ref[...])
    st = _relu(_dot(st, ws2_ref[...]) + bs2_ref[...])
    st = _dot(st, ws3_ref[...]) + bs3_ref[...]
    s = _relu(st[:, :L])
    t = st[:, L:]

    yb = (xb - t) * jnp.exp(-s)
    ldj_ref[...] = -jnp.sum(s, axis=1, keepdims=True)

    z = _relu(_dot(xa, wp_ref[:L, :]) + _dot(yb, wp_ref[L:, :]) + bp_ref[...])
    lpz_ref[...] = (-0.5 * jnp.sum(z * z, axis=1, keepdims=True)
                    - 0.5 * z.shape[1] * math.log(2.0 * math.pi))

    d = _relu(_dot(z, wd1_ref[...]) + bd1_ref[...])
    d2_scr[...] = _relu(_dot(d, wd2_ref[...]) + bd2_ref[...])

    P = K - 1
    WCi = w1b_scr.shape[1]
    Hd1 = hd_scr.shape[0]

    for io in range(Hd1):
        acc = None
        for di in range(K):
            hrow = io + di - P
            if 0 <= hrow < Ho2:
                term = _dot(d2_scr[:, hrow * WCi:(hrow + 1) * WCi], w1b_scr[di])
                acc = term if acc is None else acc + term
        hd_scr[io] = _relu(acc + db1_ref[...])

    for io in range(Hd2):
        acc = None
        for di in range(K):
            hrow = io + di - P
            if 0 <= hrow < Hd1:
                term = _dot(hd_scr[hrow], dw2_ref[di])
                acc = term if acc is None else acc + term
        dec_ref[io] = acc + db2_ref[...]


def kernel(enc_r1, enc_w1, enc_b1, enc_r2, enc_w2, enc_b2,
           pool_re, pool_ro, pool_ce, pool_co,
           fc_we, fc_be, fc_wml, fc_bml, fc_wst1, fc_bst1,
           fc_wst2, fc_bst2, fc_wst3, fc_bst3, fc_wp, fc_bp,
           fc_wd1, fc_bd1, fc_wd2, fc_bd2,
           dec_r1, dec_w1, dec_b1, dec_r2, dec_w2, dec_b2,
           x_nchw):
    B, C, H, W = x_nchw.shape
    Bc = B // _G
    K = enc_w1.shape[0]
    Ho1 = H - K + 1
    Ho2 = Ho1 - K + 1
    D_out = fc_wd2.shape[1]

    xT = jnp.transpose(x_nchw.astype(jnp.float32), (2, 0, 3, 1)).reshape(H, B, W * C)

    NF1, NF2 = enc_w1.shape[2] // (W - K + 1), enc_w2.shape[2] // (W - 2 * K + 2)
    Wo2 = W - 2 * K + 2
    c2 = jnp.concatenate(
        [enc_w2[:, w * NF1:(w + K) * NF1, w * NF2:(w + 1) * NF2]
         for w in range(Wo2)], axis=2)
    c1 = jnp.concatenate(
        [dec_w1[:, min(max(o - 2, 0), Wo2 - K) * NF2:
                (min(max(o - 2, 0), Wo2 - K) + K) * NF2, o * NF1:(o + 1) * NF1]
         for o in range(Ho1)], axis=2)

    ws = (enc_w1, enc_b1, c2, enc_b2,
          fc_we, fc_be, fc_wml, fc_bml, fc_wst1, fc_bst1,
          fc_wst2, fc_bst2, fc_wst3, fc_bst3, fc_wp, fc_bp,
          fc_wd1, fc_bd1, fc_wd2, fc_bd2,
          c1, dec_b1, dec_w2, dec_b2)

    in_specs = [pl.BlockSpec((H, Bc, W * C), lambda g: (0, g, 0))] + \
               [pl.BlockSpec(w.shape, (lambda n: (lambda g: (0,) * n))(w.ndim))
                for w in ws]

    dec, ldj, lpz = pl.pallas_call(
        _fused_kernel,
        grid=(_G,),
        out_shape=(jax.ShapeDtypeStruct((H, B, W * C), jnp.float32),
                   jax.ShapeDtypeStruct((B, 1), jnp.float32),
                   jax.ShapeDtypeStruct((B, 1), jnp.float32)),
        in_specs=in_specs,
        out_specs=(pl.BlockSpec((H, Bc, W * C), lambda g: (0, g, 0)),
                   pl.BlockSpec((Bc, 1), lambda g: (g, 0)),
                   pl.BlockSpec((Bc, 1), lambda g: (g, 0))),
        scratch_shapes=[
            pltpu.VMEM((Ho1, Bc, enc_w1.shape[2]), jnp.float32),
            pltpu.VMEM((Ho2, Bc, enc_w2.shape[2]), jnp.float32),
            pltpu.VMEM((Bc, D_out), jnp.float32),
            pltpu.VMEM((Ho1, Bc, dec_w1.shape[2]), jnp.float32),
            pltpu.VMEM(enc_w2.shape, jnp.float32),
            pltpu.VMEM(dec_w1.shape, jnp.float32),
        ],
        compiler_params=pltpu.CompilerParams(
            dimension_semantics=("parallel",), vmem_limit_bytes=_VMEM_LIMIT),
    )(xT, *ws)

    x_hat = jnp.transpose(dec.reshape(H, B, W, C), (1, 3, 0, 2))
    return x_hat, ldj[:, 0], lpz[:, 0]


# speedup vs baseline: 3.8207x; 1.7014x over previous
import math

import jax
import jax.numpy as jnp
from jax.experimental import pallas as pl
from jax.experimental.pallas import tpu as pltpu

_VMEM_LIMIT = 100 * 1024 * 1024
_BF = jnp.bfloat16


def _dot(a, b):
    return jnp.dot(a, b, preferred_element_type=jnp.float32)


def _relu(v):
    return jnp.maximum(v, 0.0)


def _fused_kernel(x_ref, w1_ref, b1_ref, c2_ref, b2_ref,
                  we_ref, be_ref, wml_ref, bml_ref,
                  ws1_ref, bs1_ref, ws2_ref, bs2_ref, ws3_ref, bs3_ref,
                  wp_ref, bp_ref, wd1_ref, bd1_ref, wd2_ref, bd2_ref,
                  c1_ref, db1_ref, dw2_ref, db2_ref,
                  dec_ref, ldj_ref, lpz_ref,
                  h1_scr, h2_scr, d2_scr, hd_scr, w2b_scr, w1b_scr):
    K = w1_ref.shape[0]
    Ho1 = h1_scr.shape[0]
    Ho2 = h2_scr.shape[0]
    NF1, NF2 = 32, 64

    w2b_scr[...] = jnp.zeros_like(w2b_scr)
    for w in range(Ho2):
        w2b_scr[:, w * NF1:(w + K) * NF1, w * NF2:(w + 1) * NF2] = \
            c2_ref[:, :, w * NF2:(w + 1) * NF2].astype(_BF)
    w1b_scr[...] = jnp.zeros_like(w1b_scr)
    for o in range(Ho1):
        s = min(max(o - 2, 0), Ho2 - K) * NF2
        w1b_scr[:, s:s + K * NF2, o * NF1:(o + 1) * NF1] = \
            c1_ref[:, :, o * NF1:(o + 1) * NF1].astype(_BF)

    for io in range(Ho1):
        acc = _dot(x_ref[io], w1_ref[0])
        for di in range(1, K):
            acc = acc + _dot(x_ref[io + di], w1_ref[di])
        h1_scr[io] = _relu(acc + b1_ref[...])

    for io in range(Ho2):
        acc = _dot(h1_scr[io].astype(_BF), w2b_scr[0])
        for di in range(1, K):
            acc = acc + _dot(h1_scr[io + di].astype(_BF), w2b_scr[di])
        h2_scr[io] = _relu(acc + b2_ref[...])

    Wp = Ho2 // 2
    pieces = []
    for i in range(Wp):
        ph = jnp.maximum(h2_scr[2 * i], h2_scr[2 * i + 1])
        for j in range(Wp):
            pieces.append(jnp.maximum(ph[:, j * 2 * NF2:(j * 2 + 1) * NF2],
                                      ph[:, (j * 2 + 1) * NF2:(j + 1) * 2 * NF2]))
    hf = jnp.concatenate(pieces, axis=1)

    L = wml_ref.shape[1] // 2

    h = _relu(_dot(hf, we_ref[...]) + be_ref[...])
    ml = _dot(h, wml_ref[...]) + bml_ref[...]
    xa, xb = ml[:, :L], ml[:, L:]

    st = _relu(_dot(xa, ws1_ref[...]) + bs1_ref[...])
    st = _relu(_dot(st, ws2_ref[...]) + bs2_ref[...])
    st = _dot(st, ws3_ref[...]) + bs3_ref[...]
    s = _relu(st[:, :L])
    t = st[:, L:]

    yb = (xb - t) * jnp.exp(-s)
    ldj_ref[...] = -jnp.sum(s, axis=1, keepdims=True)

    z = _relu(_dot(xa, wp_ref[:L, :]) + _dot(yb, wp_ref[L:, :]) + bp_ref[...])
    lpz_ref[...] = (-0.5 * jnp.sum(z * z, axis=1, keepdims=True)
                    - 0.5 * z.shape[1] * math.log(2.0 * math.pi))

    d = _relu(_dot(z, wd1_ref[...]) + bd1_ref[...])
    d2_scr[...] = _relu(_dot(d, wd2_ref[...]) + bd2_ref[...])

    P = K - 1
    WCi = w1b_scr.shape[1]
    Hd1 = hd_scr.shape[0]
    Hd2 = dec_ref.shape[0]

    for io in range(Hd1):
        acc = None
        for di in range(K):
            hrow = io + di - P
            if 0 <= hrow < Ho2:
                term = _dot(d2_scr[:, hrow * WCi:(hrow + 1) * WCi].astype(_BF),
                            w1b_scr[di])
                acc = term if acc is None else acc + term
        hd_scr[io] = _relu(acc + db1_ref[...])

    for io in range(Hd2):
        acc = None
        for di in range(K):
            hrow = io + di - P
            if 0 <= hrow < Hd1:
                term = _dot(hd_scr[hrow], dw2_ref[di])
                acc = term if acc is None else acc + term
        dec_ref[io] = acc + db2_ref[...]


def kernel(enc_r1, enc_w1, enc_b1, enc_r2, enc_w2, enc_b2,
           pool_re, pool_ro, pool_ce, pool_co,
           fc_we, fc_be, fc_wml, fc_bml, fc_wst1, fc_bst1,
           fc_wst2, fc_bst2, fc_wst3, fc_bst3, fc_wp, fc_bp,
           fc_wd1, fc_bd1, fc_wd2, fc_bd2,
           dec_r1, dec_w1, dec_b1, dec_r2, dec_w2, dec_b2,
           x_nchw):
    B, C, H, W = x_nchw.shape
    K = enc_w1.shape[0]
    Ho1 = H - K + 1
    Ho2 = Ho1 - K + 1
    D_out = fc_wd2.shape[1]

    xT = jnp.transpose(x_nchw.astype(jnp.float32), (2, 0, 3, 1)).reshape(H, B, W * C)

    NF1, NF2 = 32, 64
    c2 = jnp.concatenate(
        [enc_w2[:, w * NF1:(w + K) * NF1, w * NF2:(w + 1) * NF2]
         for w in range(Ho2)], axis=2)
    c1 = jnp.concatenate(
        [dec_w1[:, min(max(o - 2, 0), Ho2 - K) * NF2:
                (min(max(o - 2, 0), Ho2 - K) + K) * NF2, o * NF1:(o + 1) * NF1]
         for o in range(Ho1)], axis=2)

    ws = (enc_w1, enc_b1, c2, enc_b2,
          fc_we, fc_be, fc_wml, fc_bml, fc_wst1, fc_bst1,
          fc_wst2, fc_bst2, fc_wst3, fc_bst3, fc_wp, fc_bp,
          fc_wd1, fc_bd1, fc_wd2, fc_bd2,
          c1, dec_b1, dec_w2, dec_b2)

    vm = pl.BlockSpec(memory_space=pltpu.MemorySpace.VMEM)
    dec, ldj, lpz = pl.pallas_call(
        _fused_kernel,
        out_shape=(jax.ShapeDtypeStruct((H, B, W * C), jnp.float32),
                   jax.ShapeDtypeStruct((B, 1), jnp.float32),
                   jax.ShapeDtypeStruct((B, 1), jnp.float32)),
        in_specs=[vm] * (1 + len(ws)),
        out_specs=(vm, vm, vm),
        scratch_shapes=[
            pltpu.VMEM((Ho1, B, enc_w1.shape[2]), jnp.float32),
            pltpu.VMEM((Ho2, B, enc_w2.shape[2]), jnp.float32),
            pltpu.VMEM((B, D_out), jnp.float32),
            pltpu.VMEM((Ho1, B, dec_w1.shape[2]), jnp.float32),
            pltpu.VMEM(enc_w2.shape, _BF),
            pltpu.VMEM(dec_w1.shape, _BF),
        ],
        compiler_params=pltpu.CompilerParams(vmem_limit_bytes=_VMEM_LIMIT),
    )(xT, *ws)

    x_hat = jnp.transpose(dec.reshape(H, B, W, C), (1, 3, 0, 2))
    return x_hat, ldj[:, 0], lpz[:, 0]


# speedup vs baseline: 4.0627x; 1.0634x over previous
import math

import jax
import jax.numpy as jnp
from jax.experimental import pallas as pl
from jax.experimental.pallas import tpu as pltpu

_VMEM_LIMIT = 100 * 1024 * 1024
_BF = jnp.bfloat16


def _dot(a, b):
    return jnp.dot(a, b, preferred_element_type=jnp.float32)


def _relu(v):
    return jnp.maximum(v, 0.0)


def _fused_kernel(x_ref, w1_ref, b1_ref, c2_ref, b2_ref,
                  we_ref, be_ref, wml_ref, bml_ref,
                  ws1_ref, bs1_ref, ws2_ref, bs2_ref, ws3_ref, bs3_ref,
                  wp_ref, bp_ref, wd1_ref, bd1_ref, wd2_ref, bd2_ref,
                  c1_ref, db1_ref, dw2_ref, db2_ref,
                  dec_ref, ldj_ref, lpz_ref,
                  h1_scr, h2_scr, d2_scr, hd_scr, w2b_scr, w1b_scr,
                  we_scr, wd2_scr, we_sem, wd2_sem):
    K = w1_ref.shape[0]
    Ho1 = h1_scr.shape[0]
    Ho2 = h2_scr.shape[0]
    NF1, NF2 = 32, 64

    wd2_cp = pltpu.make_async_copy(wd2_ref, wd2_scr, wd2_sem)
    wd2_cp.start()
    we_cp = pltpu.make_async_copy(we_ref, we_scr, we_sem)
    we_cp.start()

    w2b_scr[...] = jnp.zeros_like(w2b_scr)
    for w in range(Ho2):
        w2b_scr[:, w * NF1:(w + K) * NF1, w * NF2:(w + 1) * NF2] = \
            c2_ref[:, :, w * NF2:(w + 1) * NF2].astype(_BF)
    w1b_scr[...] = jnp.zeros_like(w1b_scr)
    for o in range(Ho1):
        s = min(max(o - 2, 0), Ho2 - K) * NF2
        w1b_scr[:, s:s + K * NF2, o * NF1:(o + 1) * NF1] = \
            c1_ref[:, :, o * NF1:(o + 1) * NF1].astype(_BF)

    for io in range(Ho1):
        acc = _dot(x_ref[io], w1_ref[0])
        for di in range(1, K):
            acc = acc + _dot(x_ref[io + di], w1_ref[di])
        h1_scr[io] = _relu(acc + b1_ref[...])

    for io in range(Ho2):
        acc = _dot(h1_scr[io].astype(_BF), w2b_scr[0])
        for di in range(1, K):
            acc = acc + _dot(h1_scr[io + di].astype(_BF), w2b_scr[di])
        h2_scr[io] = _relu(acc + b2_ref[...])

    Wp = Ho2 // 2
    pieces = []
    for i in range(Wp):
        ph = jnp.maximum(h2_scr[2 * i], h2_scr[2 * i + 1])
        for j in range(Wp):
            pieces.append(jnp.maximum(ph[:, j * 2 * NF2:(j * 2 + 1) * NF2],
                                      ph[:, (j * 2 + 1) * NF2:(j + 1) * 2 * NF2]))
    hf = jnp.concatenate(pieces, axis=1)

    L = wml_ref.shape[1] // 2

    we_cp.wait()
    h = _relu(_dot(hf, we_scr[...]) + be_ref[...])
    ml = _dot(h, wml_ref[...]) + bml_ref[...]
    xa, xb = ml[:, :L], ml[:, L:]

    st = _relu(_dot(xa, ws1_ref[...]) + bs1_ref[...])
    st = _relu(_dot(st, ws2_ref[...]) + bs2_ref[...])
    st = _dot(st, ws3_ref[...]) + bs3_ref[...]
    s = _relu(st[:, :L])
    t = st[:, L:]

    yb = (xb - t) * jnp.exp(-s)
    ldj_ref[...] = -jnp.sum(s, axis=1, keepdims=True)

    z = _relu(_dot(xa, wp_ref[:L, :]) + _dot(yb, wp_ref[L:, :]) + bp_ref[...])
    lpz_ref[...] = (-0.5 * jnp.sum(z * z, axis=1, keepdims=True)
                    - 0.5 * z.shape[1] * math.log(2.0 * math.pi))

    d = _relu(_dot(z, wd1_ref[...]) + bd1_ref[...])
    wd2_cp.wait()
    d2_scr[...] = _relu(_dot(d, wd2_scr[...]) + bd2_ref[...])

    P = K - 1
    WCi = w1b_scr.shape[1]
    Hd1 = hd_scr.shape[0]
    Hd2 = dec_ref.shape[0]

    for io in range(Hd1):
        acc = None
        for di in range(K):
            hrow = io + di - P
            if 0 <= hrow < Ho2:
                term = _dot(d2_scr[:, hrow * WCi:(hrow + 1) * WCi].astype(_BF),
                            w1b_scr[di])
                acc = term if acc is None else acc + term
        hd_scr[io] = _relu(acc + db1_ref[...])

    for io in range(Hd2):
        acc = None
        for di in range(K):
            hrow = io + di - P
            if 0 <= hrow < Hd1:
                term = _dot(hd_scr[hrow], dw2_ref[di])
                acc = term if acc is None else acc + term
        dec_ref[io] = acc + db2_ref[...]


def kernel(enc_r1, enc_w1, enc_b1, enc_r2, enc_w2, enc_b2,
           pool_re, pool_ro, pool_ce, pool_co,
           fc_we, fc_be, fc_wml, fc_bml, fc_wst1, fc_bst1,
           fc_wst2, fc_bst2, fc_wst3, fc_bst3, fc_wp, fc_bp,
           fc_wd1, fc_bd1, fc_wd2, fc_bd2,
           dec_r1, dec_w1, dec_b1, dec_r2, dec_w2, dec_b2,
           x_nchw):
    B, C, H, W = x_nchw.shape
    K = enc_w1.shape[0]
    Ho1 = H - K + 1
    Ho2 = Ho1 - K + 1
    D_out = fc_wd2.shape[1]

    xT = jnp.transpose(x_nchw.astype(jnp.float32), (2, 0, 3, 1)).reshape(H, B, W * C)

    NF1, NF2 = 32, 64
    c2 = jnp.concatenate(
        [enc_w2[:, w * NF1:(w + K) * NF1, w * NF2:(w + 1) * NF2]
         for w in range(Ho2)], axis=2)
    c1 = jnp.concatenate(
        [dec_w1[:, min(max(o - 2, 0), Ho2 - K) * NF2:
                (min(max(o - 2, 0), Ho2 - K) + K) * NF2, o * NF1:(o + 1) * NF1]
         for o in range(Ho1)], axis=2)

    ws = (enc_w1, enc_b1, c2, enc_b2,
          fc_we, fc_be, fc_wml, fc_bml, fc_wst1, fc_bst1,
          fc_wst2, fc_bst2, fc_wst3, fc_bst3, fc_wp, fc_bp,
          fc_wd1, fc_bd1, fc_wd2, fc_bd2,
          c1, dec_b1, dec_w2, dec_b2)

    vm = pl.BlockSpec(memory_space=pltpu.MemorySpace.VMEM)
    hbm = pl.BlockSpec(memory_space=pl.ANY)
    in_specs = [vm] * (1 + len(ws))
    in_specs[1 + 4] = hbm
    in_specs[1 + 18] = hbm
    dec, ldj, lpz = pl.pallas_call(
        _fused_kernel,
        out_shape=(jax.ShapeDtypeStruct((H, B, W * C), jnp.float32),
                   jax.ShapeDtypeStruct((B, 1), jnp.float32),
                   jax.ShapeDtypeStruct((B, 1), jnp.float32)),
        in_specs=in_specs,
        out_specs=(vm, vm, vm),
        scratch_shapes=[
            pltpu.VMEM((Ho1, B, enc_w1.shape[2]), jnp.float32),
            pltpu.VMEM((Ho2, B, enc_w2.shape[2]), jnp.float32),
            pltpu.VMEM((B, D_out), jnp.float32),
            pltpu.VMEM((Ho1, B, dec_w1.shape[2]), jnp.float32),
            pltpu.VMEM(enc_w2.shape, _BF),
            pltpu.VMEM(dec_w1.shape, _BF),
            pltpu.VMEM(fc_we.shape, jnp.float32),
            pltpu.VMEM(fc_wd2.shape, jnp.float32),
            pltpu.SemaphoreType.DMA,
            pltpu.SemaphoreType.DMA,
        ],
        compiler_params=pltpu.CompilerParams(vmem_limit_bytes=_VMEM_LIMIT),
    )(xT, *ws)

    x_hat = jnp.transpose(dec.reshape(H, B, W, C), (1, 3, 0, 2))
    return x_hat, ldj[:, 0], lpz[:, 0]


# speedup vs baseline: 4.0644x; 1.0004x over previous
import math

import jax
import jax.numpy as jnp
from jax.experimental import pallas as pl
from jax.experimental.pallas import tpu as pltpu

_VMEM_LIMIT = 100 * 1024 * 1024
_BF = jnp.bfloat16


def _dot(a, b):
    return jnp.dot(a, b, preferred_element_type=jnp.float32)


def _relu(v):
    return jnp.maximum(v, 0.0)


def _fused_kernel(x_ref, w1_ref, b1_ref, c2_ref, b2_ref,
                  we_ref, be_ref, wml_ref, bml_ref,
                  ws1_ref, bs1_ref, ws2_ref, bs2_ref, ws3_ref, bs3_ref,
                  wp_ref, bp_ref, wd1_ref, bd1_ref, wd2_ref, bd2_ref,
                  c1_ref, db1_ref, dw2_ref, db2_ref,
                  dec_ref, ldj_ref, lpz_ref,
                  h1_scr, h2_scr, d2_scr, hd_scr, w2b_scr, w1b_scr,
                  we_scr, wd2_scr, we_sem, wd2_sem):
    K = w1_ref.shape[0]
    Ho1 = h1_scr.shape[0]
    Ho2 = h2_scr.shape[0]
    NF1, NF2 = 32, 64

    wd2_cp = pltpu.make_async_copy(wd2_ref, wd2_scr, wd2_sem)
    wd2_cp.start()
    we_cp = pltpu.make_async_copy(we_ref, we_scr, we_sem)
    we_cp.start()

    w2b_scr[...] = jnp.zeros_like(w2b_scr)
    for w in range(Ho2):
        w2b_scr[:, w * NF1:(w + K) * NF1, w * NF2:(w + 1) * NF2] = \
            c2_ref[:, :, w * NF2:(w + 1) * NF2].astype(_BF)
    w1b_scr[...] = jnp.zeros_like(w1b_scr)
    for o in range(Ho1):
        s = min(max(o - 2, 0), Ho2 - K) * NF2
        w1b_scr[:, s:s + K * NF2, o * NF1:(o + 1) * NF1] = \
            c1_ref[:, :, o * NF1:(o + 1) * NF1].astype(_BF)

    for io in range(Ho1):
        acc = _dot(x_ref[io], w1_ref[0])
        for di in range(1, K):
            acc = acc + _dot(x_ref[io + di], w1_ref[di])
        h1_scr[io] = _relu(acc + b1_ref[...]).astype(_BF)

    for io in range(Ho2):
        acc = _dot(h1_scr[io], w2b_scr[0])
        for di in range(1, K):
            acc = acc + _dot(h1_scr[io + di], w2b_scr[di])
        h2_scr[io] = _relu(acc + b2_ref[...])

    Wp = Ho2 // 2
    pieces = []
    for i in range(Wp):
        ph = jnp.maximum(h2_scr[2 * i], h2_scr[2 * i + 1])
        for j in range(Wp):
            pieces.append(jnp.maximum(ph[:, j * 2 * NF2:(j * 2 + 1) * NF2],
                                      ph[:, (j * 2 + 1) * NF2:(j + 1) * 2 * NF2]))
    hf = jnp.concatenate(pieces, axis=1)

    L = wml_ref.shape[1] // 2

    we_cp.wait()
    h = _relu(_dot(hf, we_scr[...]) + be_ref[...])
    ml = _dot(h, wml_ref[...]) + bml_ref[...]
    xa, xb = ml[:, :L], ml[:, L:]

    st = _relu(_dot(xa, ws1_ref[...]) + bs1_ref[...])
    st = _relu(_dot(st, ws2_ref[...]) + bs2_ref[...])
    st = _dot(st, ws3_ref[...]) + bs3_ref[...]
    s = _relu(st[:, :L])
    t = st[:, L:]

    yb = (xb - t) * jnp.exp(-s)
    ldj_ref[...] = -jnp.sum(s, axis=1, keepdims=True)

    z = _relu(_dot(xa, wp_ref[:L, :]) + _dot(yb, wp_ref[L:, :]) + bp_ref[...])
    lpz_ref[...] = (-0.5 * jnp.sum(z * z, axis=1, keepdims=True)
                    - 0.5 * z.shape[1] * math.log(2.0 * math.pi))

    d = _relu(_dot(z, wd1_ref[...]) + bd1_ref[...])
    wd2_cp.wait()
    d2_scr[...] = _relu(_dot(d, wd2_scr[...]) + bd2_ref[...]).astype(_BF)

    P = K - 1
    WCi = w1b_scr.shape[1]
    Hd1 = hd_scr.shape[0]
    Hd2 = dec_ref.shape[0]

    for io in range(Hd1):
        acc = None
        for di in range(K):
            hrow = io + di - P
            if 0 <= hrow < Ho2:
                term = _dot(d2_scr[:, hrow * WCi:(hrow + 1) * WCi],
                            w1b_scr[di])
                acc = term if acc is None else acc + term
        hd_scr[io] = _relu(acc + db1_ref[...])

    for io in range(Hd2):
        acc = None
        for di in range(K):
            hrow = io + di - P
            if 0 <= hrow < Hd1:
                term = _dot(hd_scr[hrow], dw2_ref[di])
                acc = term if acc is None else acc + term
        dec_ref[io] = acc + db2_ref[...]


def kernel(enc_r1, enc_w1, enc_b1, enc_r2, enc_w2, enc_b2,
           pool_re, pool_ro, pool_ce, pool_co,
           fc_we, fc_be, fc_wml, fc_bml, fc_wst1, fc_bst1,
           fc_wst2, fc_bst2, fc_wst3, fc_bst3, fc_wp, fc_bp,
           fc_wd1, fc_bd1, fc_wd2, fc_bd2,
           dec_r1, dec_w1, dec_b1, dec_r2, dec_w2, dec_b2,
           x_nchw):
    B, C, H, W = x_nchw.shape
    K = enc_w1.shape[0]
    Ho1 = H - K + 1
    Ho2 = Ho1 - K + 1
    D_out = fc_wd2.shape[1]

    xT = jnp.transpose(x_nchw.astype(jnp.float32), (2, 0, 3, 1)).reshape(H, B, W * C)

    NF1, NF2 = 32, 64
    c2 = jnp.concatenate(
        [enc_w2[:, w * NF1:(w + K) * NF1, w * NF2:(w + 1) * NF2]
         for w in range(Ho2)], axis=2)
    c1 = jnp.concatenate(
        [dec_w1[:, min(max(o - 2, 0), Ho2 - K) * NF2:
                (min(max(o - 2, 0), Ho2 - K) + K) * NF2, o * NF1:(o + 1) * NF1]
         for o in range(Ho1)], axis=2)

    ws = (enc_w1, enc_b1, c2, enc_b2,
          fc_we, fc_be, fc_wml, fc_bml, fc_wst1, fc_bst1,
          fc_wst2, fc_bst2, fc_wst3, fc_bst3, fc_wp, fc_bp,
          fc_wd1, fc_bd1, fc_wd2, fc_bd2,
          c1, dec_b1, dec_w2, dec_b2)

    vm = pl.BlockSpec(memory_space=pltpu.MemorySpace.VMEM)
    hbm = pl.BlockSpec(memory_space=pl.ANY)
    in_specs = [vm] * (1 + len(ws))
    in_specs[1 + 4] = hbm
    in_specs[1 + 18] = hbm
    dec, ldj, lpz = pl.pallas_call(
        _fused_kernel,
        out_shape=(jax.ShapeDtypeStruct((H, B, W * C), jnp.float32),
                   jax.ShapeDtypeStruct((B, 1), jnp.float32),
                   jax.ShapeDtypeStruct((B, 1), jnp.float32)),
        in_specs=in_specs,
        out_specs=(vm, vm, vm),
        scratch_shapes=[
            pltpu.VMEM((Ho1, B, enc_w1.shape[2]), _BF),
            pltpu.VMEM((Ho2, B, enc_w2.shape[2]), jnp.float32),
            pltpu.VMEM((B, D_out), _BF),
            pltpu.VMEM((Ho1, B, dec_w1.shape[2]), jnp.float32),
            pltpu.VMEM(enc_w2.shape, _BF),
            pltpu.VMEM(dec_w1.shape, _BF),
            pltpu.VMEM(fc_we.shape, jnp.float32),
            pltpu.VMEM(fc_wd2.shape, jnp.float32),
            pltpu.SemaphoreType.DMA,
            pltpu.SemaphoreType.DMA,
        ],
        compiler_params=pltpu.CompilerParams(vmem_limit_bytes=_VMEM_LIMIT),
    )(xT, *ws)

    x_hat = jnp.transpose(dec.reshape(H, B, W, C), (1, 3, 0, 2))
    return x_hat, ldj[:, 0], lpz[:, 0]
